# baseline (speedup 1.0000x reference)
import jax
import jax.numpy as jnp
from jax import lax
from jax.experimental import pallas as pl
from jax.experimental.pallas import tpu as pltpu
from jax.experimental.pallas import tpu_sc as plsc

N = 100000
B = 4
R = 1000000
ITERS = 5
DAMP = 0.5

NC = 2
NS = 16
L = 16

C = 768
RG = 8
CHUNKS = 164
HALF = CHUNKS // 2
PER_W = CHUNKS * C
RP = RG * PER_W

BN = B * N
DN = 2 * N
ZC = 2000
NZ = DN // ZC


def _sc_body(q_hbm, t1a, t1c, t1w, t2a, t2b, t2c, t2w, t3a, t3b, t3c, t3w,
             out_hbm, q_v,
             ia0_v, ib0_v, ic0_v, w0_v, sA0_v, mA0_v,
             ia1_v, ib1_v, ic1_v, w1_v, sA1_v, mA1_v,
             bounce_v, is0, is1, os0, os1, delta_sh):
    s = lax.axis_index("s")
    c = lax.axis_index("c")
    b = 2 * c + s // RG
    rg = s % RG
    base = rg * PER_W
    boff = (s // RG) * N

    isems = (is0, is1)
    osems = (os0, os1)
    IB = ((ia0_v, ib0_v, ic0_v), (ia1_v, ib1_v, ic1_v))
    WV = (w0_v, w1_v)
    SB = (sA0_v, sA1_v)
    MB = (mA0_v, mA1_v)

    pltpu.sync_copy(q_hbm.at[b], q_v)

    def _zv(i, carry):
        bounce_v[pl.ds(i * L, L)] = jnp.zeros((L,), jnp.float32)
        return carry

    lax.fori_loop(0, ZC // L, _zv, 0)

    def _zs(i, carry):
        k = s + i * NS

        @pl.when(k < NZ)
        def _():
            pltpu.sync_copy(bounce_v, delta_sh.at[pl.ds(k * ZC, ZC)])

        return carry

    lax.fori_loop(0, NZ // NS + 1, _zs, 0)

    plsc.subcore_barrier()

    def run_template(cols, wcol, compute):
        ncol = len(cols)

        def issue_in(ch, p):
            off = base + ch * C
            for j in range(ncol):
                pltpu.async_copy(cols[j].at[pl.ds(off, C)],
                                 IB[p][j], isems[p])
            pltpu.async_copy(wcol.at[pl.ds(off, C)], WV[p], isems[p])

        def drain_in(ch, p):
            off = base + ch * C
            for j in range(ncol):
                pltpu.make_async_copy(cols[j].at[pl.ds(off, C)],
                                      IB[p][j], isems[p]).wait()
            pltpu.make_async_copy(wcol.at[pl.ds(off, C)],
                                  WV[p], isems[p]).wait()

        def issue_out(p):
            pltpu.async_copy(MB[p], delta_sh.at[SB[p]], osems[p], add=True)

        def drain_out(p):
            pltpu.make_async_copy(MB[p], delta_sh.at[SB[p]],
                                  osems[p]).wait()

        def do_compute(p):
            @plsc.parallel_loop(0, C, step=L, unroll=4)
            def _(o):
                compute(o, p)

        if ncol == 2:
            def _fill(i, carry):
                o = 2 * C + i * L
                SB[0][pl.ds(o, L)] = jnp.zeros((L,), jnp.int32)
                SB[1][pl.ds(o, L)] = jnp.zeros((L,), jnp.int32)
                MB[0][pl.ds(o, L)] = jnp.zeros((L,), jnp.float32)
                MB[1][pl.ds(o, L)] = jnp.zeros((L,), jnp.float32)
                return carry

            lax.fori_loop(0, C // L, _fill, 0)

        issue_in(0, 0)

        def body(ch0, carry):
            for par in range(2):
                ch = 2 * ch0 + par
                if par == 0:
                    issue_in(ch + 1, 1)
                else:
                    @pl.when(ch0 < HALF - 1)
                    def _():
                        issue_in(ch + 1, 0)

                drain_in(ch, par)

                @pl.when(ch0 > 0)
                def _():
                    drain_out(par)

                do_compute(par)
                issue_out(par)
            return carry

        lax.fori_loop(0, HALF, body, 0)
        drain_out(0)
        drain_out(1)

    def compute_t1(o, p):
        ia = IB[p][0][pl.ds(o, L)]
        ic = IB[p][1][pl.ds(o, L)]
        w = WV[p][pl.ds(o, L)]
        qa = plsc.load_gather(q_v, [ia])
        qc = plsc.load_gather(q_v, [ic])
        MB[p][pl.ds(o, L)] = w * qa
        MB[p][pl.ds(C + o, L)] = w * (qc - 1.0)
        SB[p][pl.ds(o, L)] = ic + boff
        SB[p][pl.ds(C + o, L)] = ia + boff

    def compute_t2(o, p):
        ia = IB[p][0][pl.ds(o, L)]
        ib = IB[p][1][pl.ds(o, L)]
        ic = IB[p][2][pl.ds(o, L)]
        w = WV[p][pl.ds(o, L)]
        qa = plsc.load_gather(q_v, [ia])
        qb = plsc.load_gather(q_v, [ib])
        qc = plsc.load_gather(q_v, [ic])
        wqa = w * qa
        wqb = w * qb
        qc1 = qc - 1.0
        MB[p][pl.ds(o, L)] = wqa * qb
        MB[p][pl.ds(C + o, L)] = wqb * qc1
        MB[p][pl.ds(2 * C + o, L)] = wqa * qc1
        SB[p][pl.ds(o, L)] = ic + boff
        SB[p][pl.ds(C + o, L)] = ia + boff
        SB[p][pl.ds(2 * C + o, L)] = ib + boff

    def compute_t3(o, p):
        ia = IB[p][0][pl.ds(o, L)]
        ib = IB[p][1][pl.ds(o, L)]
        ic = IB[p][2][pl.ds(o, L)]
        w = WV[p][pl.ds(o, L)]
        qa = plsc.load_gather(q_v, [ia])
        qb = plsc.load_gather(q_v, [ib])
        qc = plsc.load_gather(q_v, [ic])
        qb1 = 1.0 - qb
        wqa = w * qa
        MB[p][pl.ds(o, L)] = wqa * qb1
        MB[p][pl.ds(C + o, L)] = w * qb1 * (qc - 1.0)
        MB[p][pl.ds(2 * C + o, L)] = wqa * (1.0 - qc)
        SB[p][pl.ds(o, L)] = ic + boff
        SB[p][pl.ds(C + o, L)] = ia + boff
        SB[p][pl.ds(2 * C + o, L)] = ib + boff

    run_template((t1a, t1c), t1w, compute_t1)
    run_template((t2a, t2b, t2c), t2w, compute_t2)
    run_template((t3a, t3b, t3c), t3w, compute_t3)

    plsc.subcore_barrier()

    def _wb(i, carry):
        k = s + i * NS

        @pl.when(k < NZ)
        def _():
            pltpu.sync_copy(delta_sh.at[pl.ds(k * ZC, ZC)], bounce_v)
            pltpu.sync_copy(bounce_v, out_hbm.at[pl.ds(c * DN + k * ZC, ZC)])

        return carry

    lax.fori_loop(0, NZ // NS + 1, _wb, 0)


_sc_messages = pl.kernel(
    _sc_body,
    out_type=jax.ShapeDtypeStruct((BN,), jnp.float32),
    mesh=plsc.VectorSubcoreMesh(core_axis_name="c", subcore_axis_name="s",
                                num_cores=NC, num_subcores=NS),
    compiler_params=pltpu.CompilerParams(needs_layout_passes=False),
    scratch_types=[
        pltpu.VMEM((N,), jnp.float32),
        pltpu.VMEM((C,), jnp.int32),
        pltpu.VMEM((C,), jnp.int32),
        pltpu.VMEM((C,), jnp.int32),
        pltpu.VMEM((C,), jnp.float32),
        pltpu.VMEM((3 * C,), jnp.int32),
        pltpu.VMEM((3 * C,), jnp.float32),
        pltpu.VMEM((C,), jnp.int32),
        pltpu.VMEM((C,), jnp.int32),
        pltpu.VMEM((C,), jnp.int32),
        pltpu.VMEM((C,), jnp.float32),
        pltpu.VMEM((3 * C,), jnp.int32),
        pltpu.VMEM((3 * C,), jnp.float32),
        pltpu.VMEM((ZC,), jnp.float32),
        pltpu.SemaphoreType.DMA,
        pltpu.SemaphoreType.DMA,
        pltpu.SemaphoreType.DMA,
        pltpu.SemaphoreType.DMA,
        pltpu.VMEM_SHARED((DN,), jnp.float32),
    ],
)


def _tc_update_body(cl_ref, ev_ref, d_ref, nl_ref, q_ref):
    nl = (1.0 - DAMP) * cl_ref[...] + DAMP * (ev_ref[...] + d_ref[...])
    nl_ref[...] = nl
    q_ref[...] = jax.nn.sigmoid(nl)


_tc_update = pl.pallas_call(
    _tc_update_body,
    out_shape=(jax.ShapeDtypeStruct((B, N), jnp.float32),
               jax.ShapeDtypeStruct((B, N), jnp.float32)),
)


def _tc_sigmoid_body(x_ref, o_ref):
    o_ref[...] = jax.nn.sigmoid(x_ref[...])


_tc_sigmoid = pl.pallas_call(
    _tc_sigmoid_body,
    out_shape=jax.ShapeDtypeStruct((B, N), jnp.float32),
)


def kernel(evidence_logits, t1_indices, t1_weights, t2_indices, t2_weights,
           t3_indices, t3_weights):
    pad = RP - R

    def col(a, j):
        return jnp.pad(a[:, j], (0, pad))

    t1a, t1c = col(t1_indices, 0), col(t1_indices, 1)
    t2a, t2b, t2c = col(t2_indices, 0), col(t2_indices, 1), col(t2_indices, 2)
    t3a, t3b, t3c = col(t3_indices, 0), col(t3_indices, 1), col(t3_indices, 2)
    t1wp = jnp.pad(t1_weights, (0, pad))
    t2wp = jnp.pad(t2_weights, (0, pad))
    t3wp = jnp.pad(t3_weights, (0, pad))

    curr_logits = evidence_logits
    curr_q = _tc_sigmoid(evidence_logits)
    for _ in range(ITERS):
        d = _sc_messages(curr_q, t1a, t1c, t1wp,
                         t2a, t2b, t2c, t2wp,
                         t3a, t3b, t3c, t3wp)
        curr_logits, curr_q = _tc_update(curr_logits, evidence_logits,
                                         d.reshape(B, N))
    return curr_q


# speedup vs baseline: 6.1314x; 6.1314x over previous
import jax
import jax.numpy as jnp
from jax import lax
from jax.experimental import pallas as pl
from jax.experimental.pallas import tpu as pltpu
from jax.experimental.pallas import tpu_sc as plsc

N = 100000
B = 4
R = 1000000
ITERS = 5
DAMP = 0.5

NC = 2
NS = 16
L = 16

C = 768
RG = 8
CHUNKS = 164
HALF = CHUNKS // 2
PER_W = CHUNKS * C
RP = RG * PER_W

BN = B * N
DN = 2 * N
ZC = 2000
NZ = DN // ZC


def _sc_body(q_hbm, t1a, t1c, t1w, t2a, t2b, t2c, t2w, t3a, t3b, t3c, t3w,
             out_hbm, q_v,
             ia0_v, ib0_v, ic0_v, w0_v, sa0_v, sb0_v, sc0_v,
             ma0_v, mb0_v, mc0_v,
             ia1_v, ib1_v, ic1_v, w1_v, sa1_v, sb1_v, sc1_v,
             ma1_v, mb1_v, mc1_v,
             bounce_v, is0, is1, os0, os1, delta_sh):
    s = lax.axis_index("s")
    c = lax.axis_index("c")
    b = 2 * c + s // RG
    rg = s % RG
    base = rg * PER_W
    boff = (s // RG) * N

    isems = (is0, is1)
    osems = (os0, os1)
    IB = ((ia0_v, ib0_v, ic0_v), (ia1_v, ib1_v, ic1_v))
    WV = (w0_v, w1_v)
    SB = ((sa0_v, sb0_v, sc0_v), (sa1_v, sb1_v, sc1_v))
    MB = ((ma0_v, mb0_v, mc0_v), (ma1_v, mb1_v, mc1_v))

    pltpu.sync_copy(q_hbm.at[b], q_v)

    def _zv(i, carry):
        bounce_v[pl.ds(i * L, L)] = jnp.zeros((L,), jnp.float32)
        return carry

    lax.fori_loop(0, ZC // L, _zv, 0)

    def _zs(i, carry):
        k = s + i * NS

        @pl.when(k < NZ)
        def _():
            pltpu.sync_copy(bounce_v, delta_sh.at[pl.ds(k * ZC, ZC)])

        return carry

    lax.fori_loop(0, NZ // NS + 1, _zs, 0)

    plsc.subcore_barrier()

    def run_template(cols, wcol, compute):
        ncol = len(cols)

        def issue_in(ch, p):
            off = base + ch * C
            for j in range(ncol):
                pltpu.async_copy(cols[j].at[pl.ds(off, C)],
                                 IB[p][j], isems[p])
            pltpu.async_copy(wcol.at[pl.ds(off, C)], WV[p], isems[p])

        def drain_in(ch, p):
            off = base + ch * C
            for j in range(ncol):
                pltpu.make_async_copy(cols[j].at[pl.ds(off, C)],
                                      IB[p][j], isems[p]).wait()
            pltpu.make_async_copy(wcol.at[pl.ds(off, C)],
                                  WV[p], isems[p]).wait()

        def issue_out(p):
            for j in range(ncol):
                pltpu.async_copy(MB[p][j],
                                 delta_sh.at[SB[p][j]],
                                 osems[p], add=True)

        def drain_out(p):
            for j in range(ncol):
                pltpu.make_async_copy(MB[p][j],
                                      delta_sh.at[SB[p][j]],
                                      osems[p]).wait()

        def do_compute(p):
            @plsc.parallel_loop(0, C, step=L, unroll=4)
            def _(o):
                compute(o, p)

        issue_in(0, 0)

        def body(ch0, carry):
            for par in range(2):
                ch = 2 * ch0 + par
                if par == 0:
                    issue_in(ch + 1, 1)
                else:
                    @pl.when(ch0 < HALF - 1)
                    def _():
                        issue_in(ch + 1, 0)

                drain_in(ch, par)

                @pl.when(ch0 > 0)
                def _():
                    drain_out(par)

                do_compute(par)
                issue_out(par)
            return carry

        lax.fori_loop(0, HALF, body, 0)
        drain_out(0)
        drain_out(1)

    def compute_t1(o, p):
        ia = IB[p][0][pl.ds(o, L)]
        ic = IB[p][1][pl.ds(o, L)]
        w = WV[p][pl.ds(o, L)]
        qa = plsc.load_gather(q_v, [ia])
        qc = plsc.load_gather(q_v, [ic])
        MB[p][0][pl.ds(o, L)] = w * qa
        MB[p][1][pl.ds(o, L)] = w * (qc - 1.0)
        SB[p][0][pl.ds(o, L)] = ic + boff
        SB[p][1][pl.ds(o, L)] = ia + boff

    def compute_t2(o, p):
        ia = IB[p][0][pl.ds(o, L)]
        ib = IB[p][1][pl.ds(o, L)]
        ic = IB[p][2][pl.ds(o, L)]
        w = WV[p][pl.ds(o, L)]
        qa = plsc.load_gather(q_v, [ia])
        qb = plsc.load_gather(q_v, [ib])
        qc = plsc.load_gather(q_v, [ic])
        wqa = w * qa
        wqb = w * qb
        qc1 = qc - 1.0
        MB[p][0][pl.ds(o, L)] = wqa * qb
        MB[p][1][pl.ds(o, L)] = wqb * qc1
        MB[p][2][pl.ds(o, L)] = wqa * qc1
        SB[p][0][pl.ds(o, L)] = ic + boff
        SB[p][1][pl.ds(o, L)] = ia + boff
        SB[p][2][pl.ds(o, L)] = ib + boff

    def compute_t3(o, p):
        ia = IB[p][0][pl.ds(o, L)]
        ib = IB[p][1][pl.ds(o, L)]
        ic = IB[p][2][pl.ds(o, L)]
        w = WV[p][pl.ds(o, L)]
        qa = plsc.load_gather(q_v, [ia])
        qb = plsc.load_gather(q_v, [ib])
        qc = plsc.load_gather(q_v, [ic])
        qb1 = 1.0 - qb
        wqa = w * qa
        MB[p][0][pl.ds(o, L)] = wqa * qb1
        MB[p][1][pl.ds(o, L)] = w * qb1 * (qc - 1.0)
        MB[p][2][pl.ds(o, L)] = wqa * (1.0 - qc)
        SB[p][0][pl.ds(o, L)] = ic + boff
        SB[p][1][pl.ds(o, L)] = ia + boff
        SB[p][2][pl.ds(o, L)] = ib + boff

    run_template((t1a, t1c), t1w, compute_t1)
    run_template((t2a, t2b, t2c), t2w, compute_t2)
    run_template((t3a, t3b, t3c), t3w, compute_t3)

    plsc.subcore_barrier()

    def _wb(i, carry):
        k = s + i * NS

        @pl.when(k < NZ)
        def _():
            pltpu.sync_copy(delta_sh.at[pl.ds(k * ZC, ZC)], bounce_v)
            pltpu.sync_copy(bounce_v, out_hbm.at[pl.ds(c * DN + k * ZC, ZC)])

        return carry

    lax.fori_loop(0, NZ // NS + 1, _wb, 0)


_sc_messages = pl.kernel(
    _sc_body,
    out_type=jax.ShapeDtypeStruct((BN,), jnp.float32),
    mesh=plsc.VectorSubcoreMesh(core_axis_name="c", subcore_axis_name="s",
                                num_cores=NC, num_subcores=NS),
    compiler_params=pltpu.CompilerParams(needs_layout_passes=False),
    scratch_types=[
        pltpu.VMEM((N,), jnp.float32),
        pltpu.VMEM((C,), jnp.int32),
        pltpu.VMEM((C,), jnp.int32),
        pltpu.VMEM((C,), jnp.int32),
        pltpu.VMEM((C,), jnp.float32),
        pltpu.VMEM((C,), jnp.int32),
        pltpu.VMEM((C,), jnp.int32),
        pltpu.VMEM((C,), jnp.int32),
        pltpu.VMEM((C,), jnp.float32),
        pltpu.VMEM((C,), jnp.float32),
        pltpu.VMEM((C,), jnp.float32),
        pltpu.VMEM((C,), jnp.int32),
        pltpu.VMEM((C,), jnp.int32),
        pltpu.VMEM((C,), jnp.int32),
        pltpu.VMEM((C,), jnp.float32),
        pltpu.VMEM((C,), jnp.int32),
        pltpu.VMEM((C,), jnp.int32),
        pltpu.VMEM((C,), jnp.int32),
        pltpu.VMEM((C,), jnp.float32),
        pltpu.VMEM((C,), jnp.float32),
        pltpu.VMEM((C,), jnp.float32),
        pltpu.VMEM((ZC,), jnp.float32),
        pltpu.SemaphoreType.DMA,
        pltpu.SemaphoreType.DMA,
        pltpu.SemaphoreType.DMA,
        pltpu.SemaphoreType.DMA,
        pltpu.VMEM_SHARED((DN,), jnp.float32),
    ],
)


def _tc_update_body(cl_ref, ev_ref, d_ref, nl_ref, q_ref):
    nl = (1.0 - DAMP) * cl_ref[...] + DAMP * (ev_ref[...] + d_ref[...])
    nl_ref[...] = nl
    q_ref[...] = jax.nn.sigmoid(nl)


_tc_update = pl.pallas_call(
    _tc_update_body,
    out_shape=(jax.ShapeDtypeStruct((B, N), jnp.float32),
               jax.ShapeDtypeStruct((B, N), jnp.float32)),
)


def _tc_sigmoid_body(x_ref, o_ref):
    o_ref[...] = jax.nn.sigmoid(x_ref[...])


_tc_sigmoid = pl.pallas_call(
    _tc_sigmoid_body,
    out_shape=jax.ShapeDtypeStruct((B, N), jnp.float32),
)


def kernel(evidence_logits, t1_indices, t1_weights, t2_indices, t2_weights,
           t3_indices, t3_weights):
    pad = RP - R

    def col(a, j):
        return jnp.pad(a[:, j], (0, pad))

    t1a, t1c = col(t1_indices, 0), col(t1_indices, 1)
    t2a, t2b, t2c = col(t2_indices, 0), col(t2_indices, 1), col(t2_indices, 2)
    t3a, t3b, t3c = col(t3_indices, 0), col(t3_indices, 1), col(t3_indices, 2)
    t1wp = jnp.pad(t1_weights, (0, pad))
    t2wp = jnp.pad(t2_weights, (0, pad))
    t3wp = jnp.pad(t3_weights, (0, pad))

    curr_logits = evidence_logits
    curr_q = _tc_sigmoid(evidence_logits)
    for _ in range(ITERS):
        d = _sc_messages(curr_q, t1a, t1c, t1wp,
                         t2a, t2b, t2c, t2wp,
                         t3a, t3b, t3c, t3wp)
        curr_logits, curr_q = _tc_update(curr_logits, evidence_logits,
                                         d.reshape(B, N))
    return curr_q
